# baseline (speedup 1.0000x reference)
import functools

import jax
import jax.numpy as jnp
from jax import lax
from jax.experimental import pallas as pl
from jax.experimental.pallas import tpu as pltpu
from jax.experimental.pallas import tpu_sc as plsc

N = 10000
H = 128
E = 320000
EPS = 1e-5

NP = 10240
NC = 2
NS = 16
NW = NC * NS
CHUNK = 128
NCHUNKS = 2560
EP = NCHUNKS * CHUNK
ROWS_PER_TILE = NP // NS
STG = 40
NSTAGES = 2
DEG_CH = NCHUNKS // NW

_mesh = plsc.VectorSubcoreMesh(
    core_axis_name="c", subcore_axis_name="s", num_cores=NC, num_subcores=NS)


@functools.partial(
    pl.kernel,
    out_type=jax.ShapeDtypeStruct((NC, NP, H), jnp.float32),
    mesh=_mesh,
    scratch_types=[
        pltpu.VMEM((DEG_CH, CHUNK), jnp.int32),
        pltpu.VMEM((CHUNK, H), jnp.float32),
        pltpu.VMEM((32, H), jnp.float32),
        pltpu.VMEM_SHARED((NP, H), jnp.float32),
        pltpu.SemaphoreType.DMA,
    ],
)
def _deg_kernel(dst_hbm, degp_hbm, dstbuf, ones, zbuf, dacc, sem):
    c = lax.axis_index("c")
    s = lax.axis_index("s")
    w = s * NC + c

    def fill(i, _):
        for jj in range(H // 16):
            ones[i, pl.ds(jj * 16, 16)] = jnp.ones((16,), jnp.float32)
        return 0
    lax.fori_loop(0, CHUNK, fill, 0)

    def zfill(i, _):
        for jj in range(H // 16):
            zbuf[i, pl.ds(jj * 16, 16)] = jnp.zeros((16,), jnp.float32)
        return 0
    lax.fori_loop(0, 32, zfill, 0)

    for t in range(ROWS_PER_TILE // 32):
        pltpu.sync_copy(zbuf, dacc.at[pl.ds(s * ROWS_PER_TILE + t * 32, 32)])
    pltpu.sync_copy(dst_hbm.at[pl.ds(w * DEG_CH, DEG_CH)], dstbuf)
    plsc.subcore_barrier()

    def body(j, _):
        pltpu.sync_copy(ones, dacc.at[dstbuf.at[j]], add=True)
        return 0
    lax.fori_loop(0, DEG_CH, body, 0)

    plsc.subcore_barrier()
    base = s * ROWS_PER_TILE
    pltpu.sync_copy(dacc.at[pl.ds(base, ROWS_PER_TILE)],
                    degp_hbm.at[c].at[pl.ds(base, ROWS_PER_TILE)])


@functools.partial(
    pl.kernel,
    out_type=jax.ShapeDtypeStruct((NC, NP, H), jnp.float32),
    mesh=_mesh,
    scratch_types=[
        pltpu.VMEM((STG, CHUNK), jnp.int32),
        pltpu.VMEM((STG, CHUNK), jnp.int32),
        pltpu.VMEM((CHUNK, H), jnp.float32),
        pltpu.VMEM((CHUNK, H), jnp.float32),
        pltpu.VMEM_SHARED((NP, H), jnp.float32),
        pltpu.SemaphoreType.DMA,
        pltpu.SemaphoreType.DMA,
    ],
)
def _agg_kernel(src_hbm, dst_hbm, hn_hbm, out_hbm,
                srcbuf, dstbuf, rows0, rows1, acc, semA, semS):
    c = lax.axis_index("c")
    s = lax.axis_index("s")
    w = s * NC + c

    def fill(i, _):
        for jj in range(H // 16):
            rows0[i, pl.ds(jj * 16, 16)] = jnp.zeros((16,), jnp.float32)
        return 0
    lax.fori_loop(0, CHUNK, fill, 0)

    for t in range(ROWS_PER_TILE // CHUNK):
        pltpu.sync_copy(rows0, acc.at[pl.ds(s * ROWS_PER_TILE + t * CHUNK, CHUNK)])
    plsc.subcore_barrier()

    def _gather(j, buf):
        pltpu.async_copy(hn_hbm.at[srcbuf.at[j]], buf, semA).wait()

    def _scatter(j, buf):
        pltpu.async_copy(buf, acc.at[dstbuf.at[j]], semS, add=True)

    def _drain_scatter():
        pltpu.make_async_copy(rows0, acc.at[pl.ds(0, CHUNK)], semS).wait()

    for h in range(NSTAGES):
        base = w * DEG_CH + h * STG
        pltpu.sync_copy(src_hbm.at[pl.ds(base, STG)], srcbuf)
        pltpu.sync_copy(dst_hbm.at[pl.ds(base, STG)], dstbuf)
        def lbody(j, _):
            _gather(j, rows0)
            pltpu.sync_copy(rows0, acc.at[dstbuf.at[j]], add=True)
            return 0
        lax.fori_loop(0, STG, lbody, 0)

    plsc.subcore_barrier()
    base = s * ROWS_PER_TILE
    pltpu.sync_copy(acc.at[pl.ds(base, ROWS_PER_TILE)],
                    out_hbm.at[c].at[pl.ds(base, ROWS_PER_TILE)])


RB = 1024
GRID = NP // RB


def _mm_raw_body(x_ref, w_ref, u_ref):
    u_ref[...] = jnp.dot(x_ref[...], w_ref[...],
                         preferred_element_type=jnp.float32)


_mm_raw = pl.pallas_call(
    _mm_raw_body,
    grid=(GRID,),
    in_specs=[
        pl.BlockSpec((RB, H), lambda i: (i, 0)),
        pl.BlockSpec((H, H), lambda i: (0, 0)),
    ],
    out_specs=pl.BlockSpec((RB, H), lambda i: (i, 0)),
    out_shape=jax.ShapeDtypeStruct((NP, H), jnp.float32),
)


def _scale_body(u_ref, dp_ref, hn_ref, dv_ref):
    deg = 1.0 + dp_ref[0, :, 0:1] + dp_ref[1, :, 0:1]
    dinv = lax.rsqrt(deg)
    dv_ref[...] = jnp.zeros((RB, H), jnp.float32) + dinv
    hn_ref[...] = u_ref[...] * dinv


_scale = pl.pallas_call(
    _scale_body,
    grid=(GRID,),
    in_specs=[
        pl.BlockSpec((RB, H), lambda i: (i, 0)),
        pl.BlockSpec((NC, RB, H), lambda i: (0, i, 0)),
    ],
    out_specs=[
        pl.BlockSpec((RB, H), lambda i: (i, 0)),
        pl.BlockSpec((RB, H), lambda i: (i, 0)),
    ],
    out_shape=[
        jax.ShapeDtypeStruct((NP, H), jnp.float32),
        jax.ShapeDtypeStruct((NP, H), jnp.float32),
    ],
)


def _combine_body(p_ref, hn_ref, dv_ref, b_ref, y_ref, s_ref, q_ref):
    i = pl.program_id(0)
    y = (p_ref[0] + p_ref[1] + hn_ref[...]) * dv_ref[...] + b_ref[...]
    rid = lax.broadcasted_iota(jnp.int32, (RB, 1), 0) + i * RB
    ym = jnp.where(rid < N, y, 0.0)
    y_ref[...] = ym

    @pl.when(i == 0)
    def _():
        s_ref[...] = jnp.zeros_like(s_ref)
        q_ref[...] = jnp.zeros_like(q_ref)
    s_ref[...] += jnp.sum(ym, axis=0, keepdims=True)
    q_ref[...] += jnp.sum(ym * ym, axis=0, keepdims=True)


_combine = pl.pallas_call(
    _combine_body,
    grid=(GRID,),
    in_specs=[
        pl.BlockSpec((NC, RB, H), lambda i: (0, i, 0)),
        pl.BlockSpec((RB, H), lambda i: (i, 0)),
        pl.BlockSpec((RB, H), lambda i: (i, 0)),
        pl.BlockSpec((1, H), lambda i: (0, 0)),
    ],
    out_specs=[
        pl.BlockSpec((RB, H), lambda i: (i, 0)),
        pl.BlockSpec((1, H), lambda i: (0, 0)),
        pl.BlockSpec((1, H), lambda i: (0, 0)),
    ],
    out_shape=[
        jax.ShapeDtypeStruct((NP, H), jnp.float32),
        jax.ShapeDtypeStruct((1, H), jnp.float32),
        jax.ShapeDtypeStruct((1, H), jnp.float32),
    ],
)


def _bn_relu_mm_body(y_ref, s_ref, q_ref, g_ref, be_ref, w_ref, dv_ref, hn_ref):
    i = pl.program_id(0)
    mean = s_ref[...] / N
    var = q_ref[...] / N - mean * mean
    x2 = g_ref[...] * (y_ref[...] - mean) * lax.rsqrt(var + EPS) + be_ref[...]
    x2 = jnp.maximum(x2, 0.0)
    rid = lax.broadcasted_iota(jnp.int32, (RB, 1), 0) + i * RB
    x2 = jnp.where(rid < N, x2, 0.0)
    hn_ref[...] = jnp.dot(x2, w_ref[...],
                          preferred_element_type=jnp.float32) * dv_ref[...]


_bn_relu_mm = pl.pallas_call(
    _bn_relu_mm_body,
    grid=(GRID,),
    in_specs=[
        pl.BlockSpec((RB, H), lambda i: (i, 0)),
        pl.BlockSpec((1, H), lambda i: (0, 0)),
        pl.BlockSpec((1, H), lambda i: (0, 0)),
        pl.BlockSpec((1, H), lambda i: (0, 0)),
        pl.BlockSpec((1, H), lambda i: (0, 0)),
        pl.BlockSpec((H, H), lambda i: (0, 0)),
        pl.BlockSpec((RB, H), lambda i: (i, 0)),
    ],
    out_specs=pl.BlockSpec((RB, H), lambda i: (i, 0)),
    out_shape=jax.ShapeDtypeStruct((NP, H), jnp.float32),
)


def _bn_final_body(y_ref, s_ref, q_ref, g_ref, be_ref, o_ref):
    mean = s_ref[...] / N
    var = q_ref[...] / N - mean * mean
    o_ref[...] = (g_ref[...] * (y_ref[...] - mean) * lax.rsqrt(var + EPS)
                  + be_ref[...])


_bn_final = pl.pallas_call(
    _bn_final_body,
    grid=(GRID,),
    in_specs=[
        pl.BlockSpec((RB, H), lambda i: (i, 0)),
        pl.BlockSpec((1, H), lambda i: (0, 0)),
        pl.BlockSpec((1, H), lambda i: (0, 0)),
        pl.BlockSpec((1, H), lambda i: (0, 0)),
        pl.BlockSpec((1, H), lambda i: (0, 0)),
    ],
    out_specs=pl.BlockSpec((RB, H), lambda i: (i, 0)),
    out_shape=jax.ShapeDtypeStruct((NP, H), jnp.float32),
)


def kernel(edge_index, emb, W1, b1, g1, be1, W2, b2, g2, be2):
    src = edge_index[0].astype(jnp.int32)
    dst = edge_index[1].astype(jnp.int32)
    pad = jnp.full((EP - E,), N, jnp.int32)
    src3 = jnp.concatenate([src, pad]).reshape(NCHUNKS, CHUNK)
    dst3 = jnp.concatenate([dst, pad]).reshape(NCHUNKS, CHUNK)
    emb_p = jnp.pad(emb, ((0, NP - N), (0, 0)))
    b1r, g1r, be1r = b1.reshape(1, H), g1.reshape(1, H), be1.reshape(1, H)
    b2r, g2r, be2r = b2.reshape(1, H), g2.reshape(1, H), be2.reshape(1, H)

    degp = _deg_kernel(dst3)

    u1 = _mm_raw(emb_p, W1)
    hn1, dinvb = _scale(u1, degp)
    parts1 = _agg_kernel(src3, dst3, hn1)
    y1, s1, q1 = _combine(parts1, hn1, dinvb, b1r)

    hn2 = _bn_relu_mm(y1, s1, q1, g1r, be1r, W2, dinvb)
    parts2 = _agg_kernel(src3, dst3, hn2)
    y2, s2, q2 = _combine(parts2, hn2, dinvb, b2r)

    out = _bn_final(y2, s2, q2, g2r, be2r)
    return out[:N]


# speedup vs baseline: 1.4539x; 1.4539x over previous
import functools

import jax
import jax.numpy as jnp
from jax import lax
from jax.experimental import pallas as pl
from jax.experimental.pallas import tpu as pltpu
from jax.experimental.pallas import tpu_sc as plsc

N = 10000
H = 128
E = 320000
EPS = 1e-5

NP = 10240
NC = 2
NS = 16
NW = NC * NS
CHUNK = 128
NCH = 79
EPW = NCH * CHUNK
EP = NW * EPW
ROWS_PER_TILE = NP // NS

_mesh = plsc.VectorSubcoreMesh(
    core_axis_name="c", subcore_axis_name="s", num_cores=NC, num_subcores=NS)


@functools.partial(
    pl.kernel,
    out_type=jax.ShapeDtypeStruct((NC, NP, H), jnp.float32),
    mesh=_mesh,
    scratch_types=[
        pltpu.VMEM((NCH, CHUNK), jnp.int32),
        pltpu.VMEM((CHUNK, H), jnp.float32),
        pltpu.VMEM((32, H), jnp.float32),
        pltpu.VMEM_SHARED((NP, H), jnp.float32),
        pltpu.SemaphoreType.DMA,
    ],
)
def _deg_kernel(dst_hbm, degp_hbm, dstbuf, ones, zbuf, dacc, sem):
    c = lax.axis_index("c")
    s = lax.axis_index("s")
    w = s * NC + c

    def fill(i, _):
        for jj in range(H // 16):
            ones[i, pl.ds(jj * 16, 16)] = jnp.ones((16,), jnp.float32)
        return 0
    lax.fori_loop(0, CHUNK, fill, 0)

    def zfill(i, _):
        for jj in range(H // 16):
            zbuf[i, pl.ds(jj * 16, 16)] = jnp.zeros((16,), jnp.float32)
        return 0
    lax.fori_loop(0, 32, zfill, 0)

    for t in range(ROWS_PER_TILE // 32):
        pltpu.sync_copy(zbuf, dacc.at[pl.ds(s * ROWS_PER_TILE + t * 32, 32)])
    pltpu.sync_copy(dst_hbm.at[w], dstbuf)
    plsc.subcore_barrier()

    def body(j, _):
        pltpu.sync_copy(ones, dacc.at[dstbuf.at[j]], add=True)
        return 0
    lax.fori_loop(0, NCH, body, 0)

    plsc.subcore_barrier()
    base = s * ROWS_PER_TILE
    pltpu.sync_copy(dacc.at[pl.ds(base, ROWS_PER_TILE)],
                    degp_hbm.at[c].at[pl.ds(base, ROWS_PER_TILE)])


@functools.partial(
    pl.kernel,
    out_type=jax.ShapeDtypeStruct((NC, NP, H), jnp.float32),
    mesh=_mesh,
    scratch_types=[
        pltpu.VMEM((NCH, CHUNK), jnp.int32),
        pltpu.VMEM((NCH, CHUNK), jnp.int32),
        pltpu.VMEM((CHUNK, H), jnp.float32),
        pltpu.VMEM((32, H), jnp.float32),
        pltpu.VMEM_SHARED((NP, H), jnp.float32),
        pltpu.SemaphoreType.DMA,
    ],
)
def _agg_kernel(src_hbm, dst_hbm, hn_hbm, out_hbm,
                srcbuf, dstbuf, rows, zbuf, acc, sem):
    c = lax.axis_index("c")
    s = lax.axis_index("s")
    w = s * NC + c

    def fill(i, _):
        for jj in range(H // 16):
            zbuf[i, pl.ds(jj * 16, 16)] = jnp.zeros((16,), jnp.float32)
        return 0
    lax.fori_loop(0, 32, fill, 0)

    for t in range(ROWS_PER_TILE // 32):
        pltpu.sync_copy(zbuf, acc.at[pl.ds(s * ROWS_PER_TILE + t * 32, 32)])
    pltpu.sync_copy(src_hbm.at[w], srcbuf)
    pltpu.sync_copy(dst_hbm.at[w], dstbuf)
    plsc.subcore_barrier()

    def body(j, _):
        pltpu.async_copy(hn_hbm.at[srcbuf.at[j]], rows, sem).wait()
        pltpu.sync_copy(rows, acc.at[dstbuf.at[j]], add=True)
        return 0
    lax.fori_loop(0, NCH, body, 0)

    plsc.subcore_barrier()
    base = s * ROWS_PER_TILE
    pltpu.sync_copy(acc.at[pl.ds(base, ROWS_PER_TILE)],
                    out_hbm.at[c].at[pl.ds(base, ROWS_PER_TILE)])


RB = 1024
GRID = NP // RB


def _mm_scale_body(x_ref, w_ref, dp_ref, hn_ref, dv_ref):
    deg = 1.0 + dp_ref[0, :, 0:1] + dp_ref[1, :, 0:1]
    dinv = lax.rsqrt(deg)
    dv_ref[...] = jnp.zeros((RB, H), jnp.float32) + dinv
    hn_ref[...] = jnp.dot(x_ref[...], w_ref[...],
                          preferred_element_type=jnp.float32) * dinv


_mm_scale = pl.pallas_call(
    _mm_scale_body,
    grid=(GRID,),
    in_specs=[
        pl.BlockSpec((RB, H), lambda i: (i, 0)),
        pl.BlockSpec((H, H), lambda i: (0, 0)),
        pl.BlockSpec((NC, RB, H), lambda i: (0, i, 0)),
    ],
    out_specs=[
        pl.BlockSpec((RB, H), lambda i: (i, 0)),
        pl.BlockSpec((RB, H), lambda i: (i, 0)),
    ],
    out_shape=[
        jax.ShapeDtypeStruct((NP, H), jnp.float32),
        jax.ShapeDtypeStruct((NP, H), jnp.float32),
    ],
)


def _combine_body(p_ref, hn_ref, dv_ref, b_ref, y_ref, s_ref, q_ref):
    i = pl.program_id(0)
    y = (p_ref[0] + p_ref[1] + hn_ref[...]) * dv_ref[...] + b_ref[...]
    rid = lax.broadcasted_iota(jnp.int32, (RB, 1), 0) + i * RB
    ym = jnp.where(rid < N, y, 0.0)
    y_ref[...] = ym

    @pl.when(i == 0)
    def _():
        s_ref[...] = jnp.zeros_like(s_ref)
        q_ref[...] = jnp.zeros_like(q_ref)
    s_ref[...] += jnp.sum(ym, axis=0, keepdims=True)
    q_ref[...] += jnp.sum(ym * ym, axis=0, keepdims=True)


_combine = pl.pallas_call(
    _combine_body,
    grid=(GRID,),
    in_specs=[
        pl.BlockSpec((NC, RB, H), lambda i: (0, i, 0)),
        pl.BlockSpec((RB, H), lambda i: (i, 0)),
        pl.BlockSpec((RB, H), lambda i: (i, 0)),
        pl.BlockSpec((1, H), lambda i: (0, 0)),
    ],
    out_specs=[
        pl.BlockSpec((RB, H), lambda i: (i, 0)),
        pl.BlockSpec((1, H), lambda i: (0, 0)),
        pl.BlockSpec((1, H), lambda i: (0, 0)),
    ],
    out_shape=[
        jax.ShapeDtypeStruct((NP, H), jnp.float32),
        jax.ShapeDtypeStruct((1, H), jnp.float32),
        jax.ShapeDtypeStruct((1, H), jnp.float32),
    ],
)


def _bn_relu_mm_body(y_ref, s_ref, q_ref, g_ref, be_ref, w_ref, dv_ref, hn_ref):
    i = pl.program_id(0)
    mean = s_ref[...] / N
    var = q_ref[...] / N - mean * mean
    x2 = g_ref[...] * (y_ref[...] - mean) * lax.rsqrt(var + EPS) + be_ref[...]
    x2 = jnp.maximum(x2, 0.0)
    rid = lax.broadcasted_iota(jnp.int32, (RB, 1), 0) + i * RB
    x2 = jnp.where(rid < N, x2, 0.0)
    hn_ref[...] = jnp.dot(x2, w_ref[...],
                          preferred_element_type=jnp.float32) * dv_ref[...]


_bn_relu_mm = pl.pallas_call(
    _bn_relu_mm_body,
    grid=(GRID,),
    in_specs=[
        pl.BlockSpec((RB, H), lambda i: (i, 0)),
        pl.BlockSpec((1, H), lambda i: (0, 0)),
        pl.BlockSpec((1, H), lambda i: (0, 0)),
        pl.BlockSpec((1, H), lambda i: (0, 0)),
        pl.BlockSpec((1, H), lambda i: (0, 0)),
        pl.BlockSpec((H, H), lambda i: (0, 0)),
        pl.BlockSpec((RB, H), lambda i: (i, 0)),
    ],
    out_specs=pl.BlockSpec((RB, H), lambda i: (i, 0)),
    out_shape=jax.ShapeDtypeStruct((NP, H), jnp.float32),
)


def _bn_final_body(y_ref, s_ref, q_ref, g_ref, be_ref, o_ref):
    mean = s_ref[...] / N
    var = q_ref[...] / N - mean * mean
    o_ref[...] = (g_ref[...] * (y_ref[...] - mean) * lax.rsqrt(var + EPS)
                  + be_ref[...])


_bn_final = pl.pallas_call(
    _bn_final_body,
    grid=(GRID,),
    in_specs=[
        pl.BlockSpec((RB, H), lambda i: (i, 0)),
        pl.BlockSpec((1, H), lambda i: (0, 0)),
        pl.BlockSpec((1, H), lambda i: (0, 0)),
        pl.BlockSpec((1, H), lambda i: (0, 0)),
        pl.BlockSpec((1, H), lambda i: (0, 0)),
    ],
    out_specs=pl.BlockSpec((RB, H), lambda i: (i, 0)),
    out_shape=jax.ShapeDtypeStruct((NP, H), jnp.float32),
)


def kernel(edge_index, emb, W1, b1, g1, be1, W2, b2, g2, be2):
    src = edge_index[0].astype(jnp.int32)
    dst = edge_index[1].astype(jnp.int32)
    pad = jnp.full((EP - E,), N, jnp.int32)
    src3 = jnp.concatenate([src, pad]).reshape(NW, NCH, CHUNK)
    dst3 = jnp.concatenate([dst, pad]).reshape(NW, NCH, CHUNK)
    emb_p = jnp.pad(emb, ((0, NP - N), (0, 0)))
    b1r, g1r, be1r = b1.reshape(1, H), g1.reshape(1, H), be1.reshape(1, H)
    b2r, g2r, be2r = b2.reshape(1, H), g2.reshape(1, H), be2.reshape(1, H)

    degp = _deg_kernel(dst3)

    hn1, dinvb = _mm_scale(emb_p, W1, degp)
    parts1 = _agg_kernel(src3, dst3, hn1)
    y1, s1, q1 = _combine(parts1, hn1, dinvb, b1r)

    hn2 = _bn_relu_mm(y1, s1, q1, g1r, be1r, W2, dinvb)
    parts2 = _agg_kernel(src3, dst3, hn2)
    y2, s2, q2 = _combine(parts2, hn2, dinvb, b2r)

    out = _bn_final(y2, s2, q2, g2r, be2r)
    return out[:N]
